# baseline (speedup 1.0000x reference)
import functools

import numpy as np
import jax
import jax.numpy as jnp
from jax import lax
from jax.experimental import pallas as pl
from jax.experimental.pallas import tpu as pltpu
from jax.experimental.pallas import tpu_sc as plsc

N = 10000
E = 320000
D = 128
ALPHA = 0.1
BETA = float(np.log(0.5 / 1 + 1.0))

NC = 2
NS = 16
NW = NC * NS
B = 128
NPAD = 10240
NH = N // NC
AGG = 5120
EPW = E // NW
CAP = 6144
KB = 2 * (CAP // B)
LB = CAP // B

_mesh = plsc.VectorSubcoreMesh(core_axis_name="c", subcore_axis_name="s")


@functools.partial(
    pl.kernel,
    mesh=_mesh,
    out_type=(
        jax.ShapeDtypeStruct((NW, N), jnp.float32),
        jax.ShapeDtypeStruct((NW * 2 * CAP,), jnp.int32),
        jax.ShapeDtypeStruct((NW * 2, 16), jnp.int32),
    ),
    scratch_types=[
        pltpu.VMEM((EPW,), jnp.int32),
        pltpu.VMEM((EPW,), jnp.int32),
        pltpu.VMEM((N,), jnp.float32),
        pltpu.VMEM((CAP + 16,), jnp.int32),
        pltpu.VMEM((CAP + 16,), jnp.int32),
        pltpu.VMEM((16,), jnp.int32),
        pltpu.VMEM((16,), jnp.int32),
    ],
    compiler_params=pltpu.CompilerParams(needs_layout_passes=False),
)
def _deg_kernel(src_hbm, dst_hbm, deg_hbm, lists_hbm, cnts_hbm,
                src_v, dst_v, cnt_v, l0_v, l1_v, co0_v, co1_v):
    cid = lax.axis_index("c")
    sid = lax.axis_index("s")
    wid = cid * NS + sid
    pltpu.sync_copy(src_hbm.at[wid], src_v)
    pltpu.sync_copy(dst_hbm.at[wid], dst_v)
    zeros = jnp.zeros((16,), jnp.float32)
    ones = jnp.ones((16,), jnp.float32)

    def zero_body(i, carry):
        cnt_v[pl.ds(i * 16, 16)] = zeros
        return carry

    lax.fori_loop(0, N // 16, zero_body, 0)

    def scat_body(g, offs):
        o0, o1 = offs
        sl = pl.ds(g * 16, 16)
        d16 = dst_v[sl]
        s16 = src_v[sl]
        plsc.addupdate_scatter(cnt_v, [d16], ones)
        v = s16 * 16384 + d16
        m0 = d16 < NH
        m1 = jnp.logical_not(m0)
        plsc.store_compressed(l0_v.at[pl.ds(o0[0], 16)], v, mask=m0)
        plsc.store_compressed(l1_v.at[pl.ds(o1[0], 16)], v, mask=m1)
        c0 = plsc.all_reduce_population_count(m0)
        c1 = plsc.all_reduce_population_count(m1)
        return (o0 + c0, o1 + c1)

    zi = jnp.zeros((16,), jnp.int32)
    off0, off1 = lax.fori_loop(0, EPW // 16, scat_body, (zi, zi))
    pltpu.sync_copy(cnt_v, deg_hbm.at[wid])
    pltpu.sync_copy(l0_v.at[pl.ds(0, CAP)],
                    lists_hbm.at[pl.ds((wid * 2 + 0) * CAP, CAP)])
    pltpu.sync_copy(l1_v.at[pl.ds(0, CAP)],
                    lists_hbm.at[pl.ds((wid * 2 + 1) * CAP, CAP)])
    co0_v[...] = off0
    co1_v[...] = off1
    pltpu.sync_copy(co0_v, cnts_hbm.at[wid * 2 + 0])
    pltpu.sync_copy(co1_v, cnts_hbm.at[wid * 2 + 1])


def _tc1_body(parts_ref, x_ref, y_ref, ab_ref, dinv_ref):
    deg = jnp.sum(parts_ref[...], axis=0) + 1.0
    dinv = lax.rsqrt(deg)
    x = x_ref[...]
    y_ref[0:N, :] = x * dinv[:, None]
    y_ref[N:NPAD, :] = jnp.zeros((NPAD - N, D), jnp.float32)
    q = jnp.sqrt(deg) * lax.rsqrt(jnp.sum(x * x, axis=1))
    ab_ref[...] = q[None, :]
    dinv_ref[...] = dinv[:, None]


_tc1 = pl.pallas_call(
    _tc1_body,
    out_shape=(
        jax.ShapeDtypeStruct((NPAD, D), jnp.float32),
        jax.ShapeDtypeStruct((1, N), jnp.float32),
        jax.ShapeDtypeStruct((N, 1), jnp.float32),
    ),
)


@functools.partial(
    pl.kernel,
    mesh=_mesh,
    out_type=(
        jax.ShapeDtypeStruct((NC * AGG, D), jnp.float32),
        jax.ShapeDtypeStruct((NW, 16), jnp.float32),
    ),
    scratch_types=[
        pltpu.VMEM((N,), jnp.float32),
        pltpu.VMEM((2, B), jnp.int32),
        pltpu.VMEM((2, B), jnp.int32),
        pltpu.VMEM((2, B), jnp.int32),
        pltpu.VMEM((2, B), jnp.int32),
        pltpu.VMEM((2, B), jnp.float32),
        pltpu.VMEM((2, B, D), jnp.float32),
        pltpu.VMEM((2, B, D), jnp.float32),
        pltpu.VMEM((16,), jnp.int32),
        pltpu.VMEM((16,), jnp.int32),
        pltpu.VMEM((16,), jnp.float32),
        pltpu.VMEM_SHARED((AGG, D), jnp.float32),
        pltpu.SemaphoreType.DMA,
        pltpu.SemaphoreType.DMA,
        pltpu.SemaphoreType.DMA,
        pltpu.SemaphoreType.DMA,
        pltpu.SemaphoreType.DMA,
        pltpu.SemaphoreType.DMA,
        pltpu.SemaphoreType.DMA,
        pltpu.SemaphoreType.DMA,
    ],
    compiler_params=pltpu.CompilerParams(needs_layout_passes=False),
)
def _edge_kernel(y_hbm, lists_hbm, cnts_hbm, ab_hbm, pout_hbm, cos_hbm,
                 a_v, pk_v, sg_v, dg_v, dio_v, cw_v, ys_v, yd_v,
                 cv0_v, cv1_v, ca_v, agg_sh,
                 sg0, sg1, sh0, sh1, ss0, ss1, si0, si1):
    cid = lax.axis_index("c")
    sid = lax.axis_index("s")
    wid = cid * NS + sid
    sg = (sg0, sg1)
    sh = (sh0, sh1)
    ss = (ss0, ss1)
    si = (si0, si1)
    pltpu.sync_copy(ab_hbm.at[0], a_v)
    pltpu.sync_copy(cnts_hbm.at[(2 * sid + 0) * 2 + cid], cv0_v)
    pltpu.sync_copy(cnts_hbm.at[(2 * sid + 1) * 2 + cid], cv1_v)
    cnt0 = cv0_v[...][0]
    cnt1 = cv1_v[...][0]

    zeros = jnp.zeros((16,), jnp.float32)

    def zb(i, carry):
        r = i // (D // 16)
        c = i % (D // 16)
        ys_v[0, r, pl.ds(c * 16, 16)] = zeros
        return carry

    lax.fori_loop(0, B * (D // 16), zb, 0)

    rpt = AGG // NS
    zc = 64

    def za(t, carry):
        pltpu.sync_copy(ys_v.at[0, pl.ds(0, zc)],
                        agg_sh.at[pl.ds(sid * rpt + t * zc, zc)])
        return carry

    lax.fori_loop(0, rpt // zc, za, 0)
    plsc.subcore_barrier()

    iota = lax.iota(jnp.int32, 16)
    nbase = cid * NH

    def obase(k):
        return ((2 * sid + k // LB) * 2 + cid) * CAP + (k % LB) * B

    def pk_start(k, slot):
        pltpu.async_copy(lists_hbm.at[pl.ds(obase(k), B)],
                         pk_v.at[slot], si[slot])

    def pk_wait(slot):
        pltpu.make_async_copy(lists_hbm.at[pl.ds(0, B)],
                              pk_v.at[slot], si[slot]).wait()

    def light_pass(k, slot):
        kcnt = jnp.where(k // LB == 0, cnt0, cnt1)
        for g in range(B // 16):
            sl = pl.ds(g * 16, 16)
            v = pk_v[slot, sl]
            p16 = (k % LB) * B + g * 16 + iota
            valid = p16 < kcnt
            s16 = jnp.where(valid, v >> 14, 0)
            d16 = jnp.where(valid, v & 16383, 0)
            sg_v[slot, sl] = s16
            dg_v[slot, sl] = d16
            dio_v[slot, sl] = jnp.where(valid, d16 - nbase, NH)
            q_s = plsc.load_gather(a_v, [s16])
            q_d = plsc.load_gather(a_v, [d16])
            cw_v[slot, sl] = jnp.where(valid, q_s * q_d, 0.0)

    def gather_start(slot):
        pltpu.async_copy(y_hbm.at[sg_v.at[slot]], ys_v.at[slot], sg[slot])
        pltpu.async_copy(y_hbm.at[dg_v.at[slot]], yd_v.at[slot], sh[slot])

    def gather_wait(slot):
        pltpu.make_async_copy(
            y_hbm.at[sg_v.at[slot]], ys_v.at[slot], sg[slot]).wait()
        pltpu.make_async_copy(
            y_hbm.at[dg_v.at[slot]], yd_v.at[slot], sh[slot]).wait()

    def scatter_start(slot):
        pltpu.async_copy(ys_v.at[slot], agg_sh.at[dio_v.at[slot]],
                         ss[slot], add=True)

    def scatter_wait(slot):
        pltpu.make_async_copy(
            ys_v.at[slot], agg_sh.at[dio_v.at[slot]], ss[slot]).wait()

    pk_start(0, 0)
    pk_wait(0)
    light_pass(0, 0)
    gather_start(0)
    pk_start(1, 1)

    def pair_body(i, acc):
        for b in range(2):
            k = i * 2 + b
            nxt = 1 - b

            @pl.when(jnp.logical_and(k > 0, k < KB - 1))
            def _():
                scatter_wait(nxt)

            @pl.when(k < KB - 1)
            def _():
                pk_wait(nxt)
                light_pass(k + 1, nxt)
                gather_start(nxt)

            @pl.when(k < KB - 2)
            def _():
                pk_start(k + 2, b)

            gather_wait(b)
            for g in range(B // 16):
                def edge_body(it, dot16):
                    for u in range(4):
                        e = g * 16 + it * 4 + u
                        ps = None
                        for q in range(D // 16):
                            dsl = pl.ds(q * 16, 16)
                            v1 = ys_v[b, e, dsl]
                            v2 = yd_v[b, e, dsl]
                            ps = v1 * v2 if ps is None else ps + v1 * v2
                        s = jnp.sum(ps)
                        dot16 = jnp.where(iota == it * 4 + u, s, dot16)
                    return dot16

                dot = lax.fori_loop(0, 4, edge_body, zeros)
                c16 = cw_v[b, pl.ds(g * 16, 16)]
                acc = acc + dot * c16
            scatter_start(b)
        return acc

    acc = lax.fori_loop(0, KB // 2, pair_body, zeros)
    scatter_wait(0)
    scatter_wait(1)
    ca_v[...] = acc
    pltpu.sync_copy(ca_v, cos_hbm.at[wid])
    plsc.subcore_barrier()

    def wb(t, carry):
        row = sid * rpt + t * zc
        pltpu.sync_copy(agg_sh.at[pl.ds(row, zc)],
                        pout_hbm.at[pl.ds(cid * AGG + row, zc)])
        return carry

    lax.fori_loop(0, rpt // zc, wb, 0)


def _tc2_body(p_ref, x_ref, x0_ref, dinv_ref, w_ref, lw_ref, g_ref, bb_ref,
              cos_ref, out_ref, gcd_ref):
    dinv = dinv_ref[...]
    p = jnp.concatenate(
        [p_ref[0:NH, :], p_ref[AGG:AGG + NH, :]], axis=0)
    x = x_ref[...]
    agg = dinv * p + (dinv * dinv) * x
    h = (1.0 - ALPHA) * agg + ALPHA * x0_ref[...]
    hw = lax.dot_general(h, w_ref[...], (((1,), (1,)), ((), ())),
                         preferred_element_type=jnp.float32)
    out1 = (1.0 - BETA) * h + BETA * hw
    mu = jnp.mean(out1, axis=0)
    cen = out1 - mu[None, :]
    var = jnp.mean(cen * cen, axis=0)
    o = cen * lax.rsqrt(var + 1e-5)[None, :] * g_ref[...] + bb_ref[...]
    o = jnp.maximum(o, 0.0)
    xl = lax.dot_general(x, lw_ref[...], (((1,), (1,)), ((), ())),
                         preferred_element_type=jnp.float32)
    out_ref[...] = o + xl
    gcd_ref[...] = jnp.reshape(1.0 - jnp.sum(cos_ref[...]) * (1.0 / E), (1, 1))


_tc2 = pl.pallas_call(
    _tc2_body,
    out_shape=(
        jax.ShapeDtypeStruct((N, D), jnp.float32),
        jax.ShapeDtypeStruct((1, 1), jnp.float32),
    ),
)


def kernel(x, x_0, edge_index, W, lin_w, bn_gamma, bn_beta):
    src = edge_index[0].astype(jnp.int32)
    dst = edge_index[1].astype(jnp.int32)

    deg_parts, lists, cnts = _deg_kernel(
        src.reshape(NW, EPW), dst.reshape(NW, EPW))
    y, ab, dinv = _tc1(deg_parts, x)
    pout, cosp = _edge_kernel(y, lists, cnts, ab)
    out, gcd = _tc2(pout, x, x_0, dinv, W, lin_w,
                    bn_gamma.reshape(1, D), bn_beta.reshape(1, D), cosp)
    return out, gcd.reshape(())
